# baseline (speedup 1.0000x reference)
import functools

import jax
import jax.numpy as jnp
from jax import lax
from jax.experimental import pallas as pl
from jax.experimental.pallas import tpu as pltpu
from jax.experimental.pallas import tpu_sc as plsc

_NC, _NS, _L = 2, 16, 16
_NW = _NC * _NS


@functools.lru_cache(maxsize=None)
def _make_kernel(B, D):
    b_per_w = B // _NW
    mesh = plsc.VectorSubcoreMesh(
        core_axis_name="c", subcore_axis_name="s",
        num_cores=_NC, num_subcores=_NS,
    )

    @functools.partial(
        pl.kernel,
        out_type=jax.ShapeDtypeStruct((B,), jnp.float32),
        mesh=mesh,
        scratch_types=[
            pltpu.VMEM((b_per_w,), jnp.int32),
            pltpu.VMEM((b_per_w,), jnp.int32),
            pltpu.VMEM((b_per_w, D), jnp.float32),
            pltpu.VMEM((b_per_w, D), jnp.float32),
            pltpu.VMEM((b_per_w,), jnp.float32),
            pltpu.SemaphoreType.DMA,
            pltpu.SemaphoreType.DMA,
        ],
        compiler_params=pltpu.CompilerParams(
            needs_layout_passes=False, use_tc_tiling_on_sc=False),
    )
    def k(uid_hbm, mid_hbm, ut_hbm, mt_hbm, out_hbm,
          uidx_v, midx_v, urows_v, mrows_v, res_v, sem_u, sem_m):
        wid = lax.axis_index("s") * _NC + lax.axis_index("c")
        base = wid * b_per_w
        pltpu.sync_copy(uid_hbm.at[pl.ds(base, b_per_w)], uidx_v)
        pltpu.sync_copy(mid_hbm.at[pl.ds(base, b_per_w)], midx_v)
        cu = pltpu.async_copy(ut_hbm.at[uidx_v], urows_v, sem_u)
        cm = pltpu.async_copy(mt_hbm.at[midx_v], mrows_v, sem_m)
        cu.wait()
        cm.wait()

        lane = lax.broadcasted_iota(jnp.int32, (_L,), 0)

        def body(g, carry):
            acc = jnp.zeros((_L,), jnp.float32)
            for j in range(_L):
                r = g * _L + j
                p = urows_v[r, pl.ds(0, _L)] * mrows_v[r, pl.ds(0, _L)]
                for c in range(1, D // _L):
                    p = p + (urows_v[r, pl.ds(c * _L, _L)]
                             * mrows_v[r, pl.ds(c * _L, _L)])
                s = jnp.sum(p)
                acc = jnp.where(lane == j, s, acc)
            res_v[pl.ds(g * _L, _L)] = acc
            return carry

        lax.fori_loop(0, b_per_w // _L, body, 0)
        pltpu.sync_copy(res_v, out_hbm.at[pl.ds(base, b_per_w)])

    return k


def kernel(user_id, movie_id, user_table, movie_table):
    B = user_id.shape[0]
    D = user_table.shape[1]
    out = _make_kernel(B, D)(user_id, movie_id, user_table, movie_table)
    return out.reshape(B, 1)


# speedup vs baseline: 2.1924x; 2.1924x over previous
import functools

import jax
import jax.numpy as jnp
from jax import lax
from jax.experimental import pallas as pl
from jax.experimental.pallas import tpu as pltpu
from jax.experimental.pallas import tpu_sc as plsc

_NC, _NS, _L = 2, 16, 16
_NW = _NC * _NS


@functools.lru_cache(maxsize=None)
def _make_kernel(B, D, V):
    b_per_w = B // _NW
    n_groups = b_per_w // _L
    mesh = plsc.VectorSubcoreMesh(
        core_axis_name="c", subcore_axis_name="s",
        num_cores=_NC, num_subcores=_NS,
    )

    @functools.partial(
        pl.kernel,
        out_type=jax.ShapeDtypeStruct((B,), jnp.float32),
        mesh=mesh,
        scratch_types=[
            pltpu.VMEM((b_per_w,), jnp.int32),
            pltpu.VMEM((b_per_w,), jnp.int32),
            pltpu.VMEM((D, 128), jnp.float32),
            pltpu.VMEM((D, 128), jnp.float32),
            pltpu.VMEM((D, 128), jnp.float32),
            pltpu.VMEM((D, 128), jnp.float32),
            pltpu.VMEM((b_per_w,), jnp.float32),
        ] + [pltpu.SemaphoreType.DMA] * 4,
        compiler_params=pltpu.CompilerParams(needs_layout_passes=False),
    )
    def k(uid_hbm, mid_hbm, utt_hbm, mtt_hbm, out_hbm,
          uidx_v, midx_v, bu0, bm0, bu1, bm1, res_v,
          su0, sm0, su1, sm1):
        wid = lax.axis_index("s") * _NC + lax.axis_index("c")
        base = wid * b_per_w
        pltpu.sync_copy(uid_hbm.at[pl.ds(base, b_per_w)], uidx_v)
        pltpu.sync_copy(mid_hbm.at[pl.ds(base, b_per_w)], midx_v)

        ubufs = [(bu0, su0), (bu1, su1)]
        mbufs = [(bm0, sm0), (bm1, sm1)]
        dchunks = [lax.broadcasted_iota(jnp.int32, (_L,), 0) + c * _L
                   for c in range(D // _L)]
        lane = lax.broadcasted_iota(jnp.int32, (_L,), 0)

        def body(g, carry):
            uvec = uidx_v[pl.ds(g * _L, _L)]
            mvec = midx_v[pl.ds(g * _L, _L)]
            ubase = (uvec >> 7) << 7
            mbase = (mvec >> 7) << 7
            ulane = uvec & 127
            mlane = mvec & 127

            def fire(j):
                ub, us = ubufs[j % 2]
                mb, ms = mbufs[j % 2]
                cu = pltpu.async_copy(
                    utt_hbm.at[:, pl.ds(pl.multiple_of(ubase[j], 128), 128)],
                    ub, us)
                cm = pltpu.async_copy(
                    mtt_hbm.at[:, pl.ds(pl.multiple_of(mbase[j], 128), 128)],
                    mb, ms)
                return cu, cm

            cps = [None, None]
            cps[0] = fire(0)
            cps[1] = fire(1)
            acc = jnp.zeros((_L,), jnp.float32)
            for j in range(_L):
                cu, cm = cps[j % 2]
                cu.wait()
                cm.wait()
                ub, _ = ubufs[j % 2]
                mb, _ = mbufs[j % 2]
                ucols = jnp.full((_L,), ulane[j], jnp.int32)
                mcols = jnp.full((_L,), mlane[j], jnp.int32)
                p = (plsc.load_gather(ub, [dchunks[0], ucols])
                     * plsc.load_gather(mb, [dchunks[0], mcols]))
                for c in range(1, D // _L):
                    p = p + (plsc.load_gather(ub, [dchunks[c], ucols])
                             * plsc.load_gather(mb, [dchunks[c], mcols]))
                s = jnp.sum(p)
                acc = jnp.where(lane == j, s, acc)
                if j + 2 < _L:
                    cps[j % 2] = fire(j + 2)
            res_v[pl.ds(g * _L, _L)] = acc
            return carry

        lax.fori_loop(0, n_groups, body, 0)
        pltpu.sync_copy(res_v, out_hbm.at[pl.ds(base, b_per_w)])

    return k


def kernel(user_id, movie_id, user_table, movie_table):
    B = user_id.shape[0]
    V, D = user_table.shape
    out = _make_kernel(B, D, V)(user_id, movie_id, user_table.T,
                                movie_table.T)
    return out.reshape(B, 1)


# speedup vs baseline: 2.4404x; 1.1131x over previous
import functools

import jax
import jax.numpy as jnp
from jax import lax
from jax.experimental import pallas as pl
from jax.experimental.pallas import tpu as pltpu
from jax.experimental.pallas import tpu_sc as plsc

_NC, _NS, _L = 2, 16, 16
_NW = _NC * _NS
_CH = 4
_NST = 8


@functools.lru_cache(maxsize=None)
def _make_extract(B, D, V):
    tcn = (V + 127) // 128
    mesh = plsc.VectorSubcoreMesh(
        core_axis_name="c", subcore_axis_name="s",
        num_cores=_NC, num_subcores=_NS,
    )

    @functools.partial(
        pl.kernel,
        out_type=(jax.ShapeDtypeStruct((B * D,), jnp.float32),
                  jax.ShapeDtypeStruct((B * D,), jnp.float32)),
        mesh=mesh,
        scratch_types=[
            pltpu.VMEM((B,), jnp.int32),
            pltpu.VMEM((B + _L,), jnp.int32),
            pltpu.VMEM((B + _L,), jnp.int32),
            pltpu.VMEM((D, _CH * 128), jnp.float32),
            pltpu.VMEM((_L,), jnp.int32),
            pltpu.VMEM((_L,), jnp.int32),
            pltpu.VMEM((_NST * D,), jnp.float32),
            pltpu.SMEM((1,), jnp.int32),
            pltpu.SMEM((1,), jnp.int32),
            pltpu.SemaphoreType.DMA,
            pltpu.SemaphoreType.DMA,
        ],
        compiler_params=pltpu.CompilerParams(needs_layout_passes=False),
    )
    def k(uid_hbm, mid_hbm, utt_hbm, mtt_hbm, uscr_hbm, mscr_hbm,
          ids_v, lid_v, lpos_v, buf_v, hcol_v, hpos_v, stage_v,
          cnt_s, iss_s, sem_c, sem_w):
        wid = lax.axis_index("s") * _NC + lax.axis_index("c")
        lo = (wid * tcn) // _NW
        hi = ((wid + 1) * tcn) // _NW
        lane = lax.broadcasted_iota(jnp.int32, (_L,), 0)
        dchunks = [lane + c * _L for c in range(D // _L)]
        iss_s[0] = 0

        def phase(id_hbm, tab_hbm, scr_hbm):
            pltpu.sync_copy(id_hbm, ids_v)
            cnt_s[0] = 0

            def scan(g, carry):
                idv = ids_v[pl.ds(g * _L, _L)]
                tcv = idv >> 7
                m = (tcv >= lo) & (tcv < hi)
                cnt = cnt_s[0]
                plsc.store_compressed(lid_v.at[pl.ds(cnt, _L)], idv, mask=m)
                pos = lane + g * _L
                plsc.store_compressed(lpos_v.at[pl.ds(cnt, _L)], pos, mask=m)
                n = plsc.all_reduce_population_count(m)
                cnt_s[0] = cnt + n[0]
                return carry

            lax.fori_loop(0, B // _L, scan, 0)
            cnt = cnt_s[0]
            ngr = (cnt + _L - 1) // _L
            nch = (hi - lo + _CH - 1) // _CH

            def chunk(cc, carry):
                tc0 = lo + cc * _CH
                cb = pl.multiple_of(tc0 << 7, 128)
                pltpu.async_copy(tab_hbm.at[:, pl.ds(cb, _CH * 128)],
                                 buf_v, sem_c).wait()

                def group(gg, carry2):
                    lv = lid_v[pl.ds(gg * _L, _L)]
                    pv = lpos_v[pl.ds(gg * _L, _L)]
                    rel = (lv >> 7) - tc0
                    m2 = (rel >= 0) & (rel < _CH) & ((gg * _L + lane) < cnt)
                    colv = lv - (tc0 << 7)
                    plsc.store_compressed(hcol_v.at[pl.ds(0, _L)], colv, mask=m2)
                    plsc.store_compressed(hpos_v.at[pl.ds(0, _L)], pv, mask=m2)
                    nh = plsc.all_reduce_population_count(m2)[0]

                    def hit(h, carry3):
                        hsplat = jnp.full((_L,), h, jnp.int32)
                        colsp = plsc.load_gather(hcol_v, [hsplat])
                        possp = plsc.load_gather(hpos_v, [hsplat])
                        pos = possp[0]
                        iss = iss_s[0]
                        soff = pl.multiple_of((iss % _NST) * D, 8)
                        for c in range(D // _L):
                            v = plsc.load_gather(buf_v, [dchunks[c], colsp])
                            stage_v[pl.ds(soff + c * _L, _L)] = v
                        @pl.when(iss >= _NST)
                        def _():
                            pltpu.make_async_copy(
                                scr_hbm.at[pl.ds(0, D)],
                                stage_v.at[pl.ds(0, D)], sem_w).wait()
                        pltpu.async_copy(
                            stage_v.at[pl.ds(soff, D)],
                            scr_hbm.at[pl.ds(pos * D, D)], sem_w)
                        iss_s[0] = iss + 1
                        return carry3

                    lax.fori_loop(0, nh, hit, 0)
                    return carry2

                lax.fori_loop(0, ngr, group, 0)
                return carry

            lax.fori_loop(0, nch, chunk, 0)

        phase(uid_hbm, utt_hbm, uscr_hbm)
        phase(mid_hbm, mtt_hbm, mscr_hbm)

        rem = jnp.minimum(iss_s[0], _NST)

        def drain(i, carry):
            pltpu.make_async_copy(uscr_hbm.at[pl.ds(0, D)],
                                  stage_v.at[pl.ds(0, D)], sem_w).wait()
            return carry

        lax.fori_loop(0, rem, drain, 0)

    return k


@functools.lru_cache(maxsize=None)
def _make_dot(B, D):
    b_per_w = B // _NW
    n_groups = b_per_w // _L
    mesh = plsc.VectorSubcoreMesh(
        core_axis_name="c", subcore_axis_name="s",
        num_cores=_NC, num_subcores=_NS,
    )

    @functools.partial(
        pl.kernel,
        out_type=jax.ShapeDtypeStruct((B,), jnp.float32),
        mesh=mesh,
        scratch_types=[
            pltpu.VMEM((b_per_w * D,), jnp.float32),
            pltpu.VMEM((b_per_w * D,), jnp.float32),
            pltpu.VMEM((b_per_w,), jnp.float32),
            pltpu.SemaphoreType.DMA,
            pltpu.SemaphoreType.DMA,
        ],
        compiler_params=pltpu.CompilerParams(needs_layout_passes=False),
    )
    def k(uscr_hbm, mscr_hbm, out_hbm, ubuf_v, mbuf_v, res_v, sem_u, sem_m):
        wid = lax.axis_index("s") * _NC + lax.axis_index("c")
        base = wid * b_per_w
        cu = pltpu.async_copy(uscr_hbm.at[pl.ds(base * D, b_per_w * D)],
                              ubuf_v, sem_u)
        cm = pltpu.async_copy(mscr_hbm.at[pl.ds(base * D, b_per_w * D)],
                              mbuf_v, sem_m)
        cu.wait()
        cm.wait()
        lane = lax.broadcasted_iota(jnp.int32, (_L,), 0)

        def body(g, carry):
            acc = jnp.zeros((_L,), jnp.float32)
            for j in range(_L):
                r = (g * _L + j) * D
                p = (ubuf_v[pl.ds(r, _L)] * mbuf_v[pl.ds(r, _L)])
                for c in range(1, D // _L):
                    p = p + (ubuf_v[pl.ds(r + c * _L, _L)]
                             * mbuf_v[pl.ds(r + c * _L, _L)])
                s = jnp.sum(p)
                acc = jnp.where(lane == j, s, acc)
            res_v[pl.ds(g * _L, _L)] = acc
            return carry

        lax.fori_loop(0, n_groups, body, 0)
        pltpu.sync_copy(res_v, out_hbm.at[pl.ds(base, b_per_w)])

    return k


def kernel(user_id, movie_id, user_table, movie_table):
    B = user_id.shape[0]
    V, D = user_table.shape
    uscr, mscr = _make_extract(B, D, V)(user_id, movie_id,
                                        user_table.T, movie_table.T)
    out = _make_dot(B, D)(uscr, mscr)
    return out.reshape(B, 1)


# speedup vs baseline: 3.6304x; 1.4876x over previous
import functools

import jax
import jax.numpy as jnp
from jax import lax
from jax.experimental import pallas as pl
from jax.experimental.pallas import tpu as pltpu
from jax.experimental.pallas import tpu_sc as plsc

_NC, _NS, _L = 2, 16, 16
_NW = _NC * _NS
_CH = 4
_NST = 8


@functools.lru_cache(maxsize=None)
def _make_extract(B, D, V):
    tcn = (V + 127) // 128
    mesh = plsc.VectorSubcoreMesh(
        core_axis_name="c", subcore_axis_name="s",
        num_cores=_NC, num_subcores=_NS,
    )

    @functools.partial(
        pl.kernel,
        out_type=(jax.ShapeDtypeStruct((B * D,), jnp.float32),
                  jax.ShapeDtypeStruct((B * D,), jnp.float32)),
        mesh=mesh,
        scratch_types=[
            pltpu.VMEM((B,), jnp.int32),
            pltpu.VMEM((B + _L,), jnp.int32),
            pltpu.VMEM((B + _L,), jnp.int32),
            pltpu.VMEM((D, _CH * 128), jnp.float32),
            pltpu.VMEM((D, _CH * 128), jnp.float32),
            pltpu.VMEM((_L,), jnp.int32),
            pltpu.VMEM((_L,), jnp.int32),
            pltpu.VMEM((_NST * D,), jnp.float32),
            pltpu.SMEM((1,), jnp.int32),
            pltpu.SMEM((1,), jnp.int32),
            pltpu.SemaphoreType.DMA,
            pltpu.SemaphoreType.DMA,
            pltpu.SemaphoreType.DMA,
        ],
        compiler_params=pltpu.CompilerParams(needs_layout_passes=False),
    )
    def k(uid_hbm, mid_hbm, utt_hbm, mtt_hbm, uscr_hbm, mscr_hbm,
          ids_v, lid_v, lpos_v, bufa_v, bufb_v, hcol_v, hpos_v, stage_v,
          cnt_s, iss_s, sem_a, sem_b, sem_w):
        wid = lax.axis_index("s") * _NC + lax.axis_index("c")
        lo = (wid * tcn) // _NW
        hi = ((wid + 1) * tcn) // _NW
        lane = lax.broadcasted_iota(jnp.int32, (_L,), 0)
        dchunks = [lane + c * _L for c in range(D // _L)]
        iss_s[0] = 0

        def phase(id_hbm, tab_hbm, scr_hbm):
            pltpu.sync_copy(id_hbm, ids_v)
            cnt_s[0] = 0

            def scan(g, carry):
                idv = ids_v[pl.ds(g * _L, _L)]
                tcv = idv >> 7
                m = (tcv >= lo) & (tcv < hi)
                cnt = cnt_s[0]
                plsc.store_compressed(lid_v.at[pl.ds(cnt, _L)], idv, mask=m)
                pos = lane + g * _L
                plsc.store_compressed(lpos_v.at[pl.ds(cnt, _L)], pos, mask=m)
                n = plsc.all_reduce_population_count(m)
                cnt_s[0] = cnt + n[0]
                return carry

            lax.fori_loop(0, B // _L, scan, 0)
            cnt = cnt_s[0]
            ngr = (cnt + _L - 1) // _L
            nch = (hi - lo + _CH - 1) // _CH

            def fire(cc, buf, sem):
                tc0 = lo + cc * _CH
                cb = pl.multiple_of(tc0 << 7, 128)
                return pltpu.async_copy(tab_hbm.at[:, pl.ds(cb, _CH * 128)],
                                        buf, sem)

            def process(cc, buf_v):
                tc0 = lo + cc * _CH

                def group(gg, carry2):
                    lv = lid_v[pl.ds(gg * _L, _L)]
                    pv = lpos_v[pl.ds(gg * _L, _L)]
                    rel = (lv >> 7) - tc0
                    m2 = (rel >= 0) & (rel < _CH) & ((gg * _L + lane) < cnt)
                    colv = lv - (tc0 << 7)
                    plsc.store_compressed(hcol_v.at[pl.ds(0, _L)], colv, mask=m2)
                    plsc.store_compressed(hpos_v.at[pl.ds(0, _L)], pv, mask=m2)
                    nh = plsc.all_reduce_population_count(m2)[0]

                    def hit(h, carry3):
                        hsplat = jnp.full((_L,), h, jnp.int32)
                        colsp = plsc.load_gather(hcol_v, [hsplat])
                        possp = plsc.load_gather(hpos_v, [hsplat])
                        pos = possp[0]
                        iss = iss_s[0]
                        soff = pl.multiple_of((iss % _NST) * D, 8)
                        for c in range(D // _L):
                            v = plsc.load_gather(buf_v, [dchunks[c], colsp])
                            stage_v[pl.ds(soff + c * _L, _L)] = v
                        @pl.when(iss >= _NST)
                        def _():
                            pltpu.make_async_copy(
                                scr_hbm.at[pl.ds(0, D)],
                                stage_v.at[pl.ds(0, D)], sem_w).wait()
                        pltpu.async_copy(
                            stage_v.at[pl.ds(soff, D)],
                            scr_hbm.at[pl.ds(pos * D, D)], sem_w)
                        iss_s[0] = iss + 1
                        return carry3

                    lax.fori_loop(0, nh, hit, 0)
                    return carry2

                lax.fori_loop(0, ngr, group, 0)

            def wait(buf, sem):
                pltpu.make_async_copy(
                    tab_hbm.at[:, pl.ds(0, _CH * 128)], buf, sem).wait()

            fire(0, bufa_v, sem_a)
            nit = (nch + 1) // 2

            def it_body(it, carry):
                cc0 = it * 2
                cc1 = cc0 + 1
                wait(bufa_v, sem_a)

                @pl.when(cc1 < nch)
                def _():
                    fire(cc1, bufb_v, sem_b)

                process(cc0, bufa_v)

                @pl.when(cc1 < nch)
                def _():
                    wait(bufb_v, sem_b)

                    @pl.when(cc1 + 1 < nch)
                    def _():
                        fire(cc1 + 1, bufa_v, sem_a)

                    process(cc1, bufb_v)

                return carry

            lax.fori_loop(0, nit, it_body, 0)

        phase(uid_hbm, utt_hbm, uscr_hbm)
        phase(mid_hbm, mtt_hbm, mscr_hbm)

        rem = jnp.minimum(iss_s[0], _NST)

        def drain(i, carry):
            pltpu.make_async_copy(uscr_hbm.at[pl.ds(0, D)],
                                  stage_v.at[pl.ds(0, D)], sem_w).wait()
            return carry

        lax.fori_loop(0, rem, drain, 0)

    return k


@functools.lru_cache(maxsize=None)
def _make_dot(B, D):
    b_per_w = B // _NW
    n_groups = b_per_w // _L
    mesh = plsc.VectorSubcoreMesh(
        core_axis_name="c", subcore_axis_name="s",
        num_cores=_NC, num_subcores=_NS,
    )

    @functools.partial(
        pl.kernel,
        out_type=jax.ShapeDtypeStruct((B,), jnp.float32),
        mesh=mesh,
        scratch_types=[
            pltpu.VMEM((b_per_w * D,), jnp.float32),
            pltpu.VMEM((b_per_w * D,), jnp.float32),
            pltpu.VMEM((b_per_w,), jnp.float32),
            pltpu.SemaphoreType.DMA,
            pltpu.SemaphoreType.DMA,
        ],
        compiler_params=pltpu.CompilerParams(needs_layout_passes=False),
    )
    def k(uscr_hbm, mscr_hbm, out_hbm, ubuf_v, mbuf_v, res_v, sem_u, sem_m):
        wid = lax.axis_index("s") * _NC + lax.axis_index("c")
        base = wid * b_per_w
        cu = pltpu.async_copy(uscr_hbm.at[pl.ds(base * D, b_per_w * D)],
                              ubuf_v, sem_u)
        cm = pltpu.async_copy(mscr_hbm.at[pl.ds(base * D, b_per_w * D)],
                              mbuf_v, sem_m)
        cu.wait()
        cm.wait()
        lane = lax.broadcasted_iota(jnp.int32, (_L,), 0)

        def body(g, carry):
            acc = jnp.zeros((_L,), jnp.float32)
            for j in range(_L):
                r = (g * _L + j) * D
                p = (ubuf_v[pl.ds(r, _L)] * mbuf_v[pl.ds(r, _L)])
                for c in range(1, D // _L):
                    p = p + (ubuf_v[pl.ds(r + c * _L, _L)]
                             * mbuf_v[pl.ds(r + c * _L, _L)])
                s = jnp.sum(p)
                acc = jnp.where(lane == j, s, acc)
            res_v[pl.ds(g * _L, _L)] = acc
            return carry

        lax.fori_loop(0, n_groups, body, 0)
        pltpu.sync_copy(res_v, out_hbm.at[pl.ds(base, b_per_w)])

    return k


def kernel(user_id, movie_id, user_table, movie_table):
    B = user_id.shape[0]
    V, D = user_table.shape
    uscr, mscr = _make_extract(B, D, V)(user_id, movie_id,
                                        user_table.T, movie_table.T)
    out = _make_dot(B, D)(uscr, mscr)
    return out.reshape(B, 1)


# speedup vs baseline: 4.6796x; 1.2890x over previous
import functools

import jax
import jax.numpy as jnp
from jax import lax
from jax.experimental import pallas as pl
from jax.experimental.pallas import tpu as pltpu
from jax.experimental.pallas import tpu_sc as plsc

_NC, _NS, _L = 2, 16, 16
_NW = _NC * _NS
_CH = 4
_NST = 8


@functools.lru_cache(maxsize=None)
def _make_extract(B, D, V):
    tcn = (V + 127) // 128
    mesh = plsc.VectorSubcoreMesh(
        core_axis_name="c", subcore_axis_name="s",
        num_cores=_NC, num_subcores=_NS,
    )

    @functools.partial(
        pl.kernel,
        out_type=(jax.ShapeDtypeStruct((B * D,), jnp.float32),
                  jax.ShapeDtypeStruct((B * D,), jnp.float32)),
        mesh=mesh,
        scratch_types=[
            pltpu.VMEM((2048,), jnp.int32),
            pltpu.VMEM((B + _L,), jnp.int32),
            pltpu.VMEM((D, _CH * 128), jnp.float32),
            pltpu.VMEM((D, _CH * 128), jnp.float32),
            pltpu.VMEM((D, _CH * 128), jnp.float32),
            pltpu.VMEM((_L,), jnp.int32),
            pltpu.VMEM((_NST * D,), jnp.float32),
            pltpu.SMEM((1,), jnp.int32),
            pltpu.SMEM((1,), jnp.int32),
            pltpu.SemaphoreType.DMA,
            pltpu.SemaphoreType.DMA,
            pltpu.SemaphoreType.DMA,
            pltpu.SemaphoreType.DMA,
        ],
        compiler_params=pltpu.CompilerParams(needs_layout_passes=False),
    )
    def k(uid_hbm, mid_hbm, utt_hbm, mtt_hbm, uscr_hbm, mscr_hbm,
          ids_v, lpk_v, bufa_v, bufb_v, bufc_v, hpk_v, stage_v,
          cnt_s, iss_s, sem_a, sem_b, sem_c, sem_w):
        wid = lax.axis_index("s") * _NC + lax.axis_index("c")
        lo = (wid * tcn) // _NW
        hi = ((wid + 1) * tcn) // _NW
        lane = lax.broadcasted_iota(jnp.int32, (_L,), 0)
        dchunks = [lane + c * _L for c in range(D // _L)]
        iss_s[0] = 0
        bufs = [(bufa_v, sem_a), (bufb_v, sem_b), (bufc_v, sem_c)]

        def phase(id_hbm, tab_hbm, scr_hbm):
            nch = (hi - lo + _CH - 1) // _CH

            def fire(cc, buf, sem):
                tc0 = lo + cc * _CH
                cb = pl.multiple_of(tc0 << 7, 128)
                return pltpu.async_copy(tab_hbm.at[:, pl.ds(cb, _CH * 128)],
                                        buf, sem)

            fire(0, bufa_v, sem_a)

            @pl.when(nch > 1)
            def _():
                fire(1, bufb_v, sem_b)

            cnt_s[0] = 0

            def slice_scan(sl, carry):
                pltpu.sync_copy(id_hbm.at[pl.ds(sl * 2048, 2048)], ids_v)

                def scan(g, carry2):
                    idv = ids_v[pl.ds(g * _L, _L)]
                    tcv = idv >> 7
                    m = (tcv >= lo) & (tcv < hi)
                    cnt = cnt_s[0]
                    pos = lane + (sl * 2048 + g * _L)
                    packv = (((tcv - lo) << 21) | ((idv & 127) << 14) | pos)
                    plsc.store_compressed(lpk_v.at[pl.ds(cnt, _L)], packv,
                                          mask=m)
                    n = plsc.all_reduce_population_count(m)
                    cnt_s[0] = cnt + n[0]
                    return carry2

                lax.fori_loop(0, 2048 // _L, scan, 0)
                return carry

            lax.fori_loop(0, B // 2048, slice_scan, 0)
            cnt = cnt_s[0]
            ngr = (cnt + _L - 1) // _L

            def process(cc, buf_v):
                rel0 = cc * _CH

                def group(gg, carry2):
                    lv = lpk_v[pl.ds(gg * _L, _L)]
                    rel = (lv >> 21) - rel0
                    m2 = (rel >= 0) & (rel < _CH) & ((gg * _L + lane) < cnt)
                    nh = plsc.all_reduce_population_count(m2)[0]

                    @pl.when(nh > 0)
                    def _():
                        plsc.store_compressed(hpk_v.at[pl.ds(0, _L)], lv,
                                              mask=m2)

                        def hit(h, carry3):
                            hsplat = jnp.full((_L,), h, jnp.int32)
                            pk = plsc.load_gather(hpk_v, [hsplat])
                            colsp = (((pk >> 21) - rel0) << 7) | ((pk >> 14)
                                                                  & 127)
                            pos = (pk & 16383)[0]
                            iss = iss_s[0]
                            soff = pl.multiple_of((iss % _NST) * D, 8)
                            for c in range(D // _L):
                                v = plsc.load_gather(buf_v,
                                                     [dchunks[c], colsp])
                                stage_v[pl.ds(soff + c * _L, _L)] = v
                            @pl.when(iss >= _NST)
                            def _():
                                pltpu.make_async_copy(
                                    scr_hbm.at[pl.ds(0, D)],
                                    stage_v.at[pl.ds(0, D)], sem_w).wait()
                            pltpu.async_copy(
                                stage_v.at[pl.ds(soff, D)],
                                scr_hbm.at[pl.ds(pos * D, D)], sem_w)
                            iss_s[0] = iss + 1
                            return carry3

                        lax.fori_loop(0, nh, hit, 0)

                    return carry2

                lax.fori_loop(0, ngr, group, 0)

            def waitc(buf, sem):
                pltpu.make_async_copy(
                    tab_hbm.at[:, pl.ds(0, _CH * 128)], buf, sem).wait()

            nit = (nch + 2) // 3

            def it_body(it, carry):
                for q in range(3):
                    cc = it * 3 + q
                    buf, sem = bufs[q]
                    nbuf, nsem = bufs[(q + 2) % 3]

                    @pl.when(cc < nch)
                    def _():
                        waitc(buf, sem)

                        @pl.when(cc + 2 < nch)
                        def _():
                            fire(cc + 2, nbuf, nsem)

                        process(cc, buf)

                return carry

            lax.fori_loop(0, nit, it_body, 0)

        phase(uid_hbm, utt_hbm, uscr_hbm)
        phase(mid_hbm, mtt_hbm, mscr_hbm)

        rem = jnp.minimum(iss_s[0], _NST)

        def drain(i, carry):
            pltpu.make_async_copy(uscr_hbm.at[pl.ds(0, D)],
                                  stage_v.at[pl.ds(0, D)], sem_w).wait()
            return carry

        lax.fori_loop(0, rem, drain, 0)

    return k


@functools.lru_cache(maxsize=None)
def _make_dot(B, D):
    b_per_w = B // _NW
    n_groups = b_per_w // _L
    mesh = plsc.VectorSubcoreMesh(
        core_axis_name="c", subcore_axis_name="s",
        num_cores=_NC, num_subcores=_NS,
    )

    @functools.partial(
        pl.kernel,
        out_type=jax.ShapeDtypeStruct((B,), jnp.float32),
        mesh=mesh,
        scratch_types=[
            pltpu.VMEM((b_per_w * D,), jnp.float32),
            pltpu.VMEM((b_per_w * D,), jnp.float32),
            pltpu.VMEM((b_per_w,), jnp.float32),
            pltpu.SemaphoreType.DMA,
            pltpu.SemaphoreType.DMA,
        ],
        compiler_params=pltpu.CompilerParams(needs_layout_passes=False),
    )
    def k(uscr_hbm, mscr_hbm, out_hbm, ubuf_v, mbuf_v, res_v, sem_u, sem_m):
        wid = lax.axis_index("s") * _NC + lax.axis_index("c")
        base = wid * b_per_w
        cu = pltpu.async_copy(uscr_hbm.at[pl.ds(base * D, b_per_w * D)],
                              ubuf_v, sem_u)
        cm = pltpu.async_copy(mscr_hbm.at[pl.ds(base * D, b_per_w * D)],
                              mbuf_v, sem_m)
        cu.wait()
        cm.wait()
        lane = lax.broadcasted_iota(jnp.int32, (_L,), 0)

        def body(g, carry):
            acc = jnp.zeros((_L,), jnp.float32)
            for j in range(_L):
                r = (g * _L + j) * D
                p = (ubuf_v[pl.ds(r, _L)] * mbuf_v[pl.ds(r, _L)])
                for c in range(1, D // _L):
                    p = p + (ubuf_v[pl.ds(r + c * _L, _L)]
                             * mbuf_v[pl.ds(r + c * _L, _L)])
                s = jnp.sum(p)
                acc = jnp.where(lane == j, s, acc)
            res_v[pl.ds(g * _L, _L)] = acc
            return carry

        lax.fori_loop(0, n_groups, body, 0)
        pltpu.sync_copy(res_v, out_hbm.at[pl.ds(base, b_per_w)])

    return k


def kernel(user_id, movie_id, user_table, movie_table):
    B = user_id.shape[0]
    V, D = user_table.shape
    uscr, mscr = _make_extract(B, D, V)(user_id, movie_id,
                                        user_table.T, movie_table.T)
    out = _make_dot(B, D)(uscr, mscr)
    return out.reshape(B, 1)
